# baseline (speedup 1.0000x reference)
import functools

import jax
import jax.numpy as jnp
from jax.experimental import pallas as pl
from jax.experimental.pallas import tpu as pltpu

B = 32
D = 128
FF = 512
DEPTH = 7
MEM_LAYER = 6
M = 8192
K = 32
P = 16
G = 224 // P
NT = G * G
NC = 1000
BB = 2


def _ln(x, g, b):
    m = x.mean(-1, keepdims=True)
    v = ((x - m) ** 2).mean(-1, keepdims=True)
    return (x - m) * jax.lax.rsqrt(v + 1e-5) * g + b


def _dot(a, b):
    return jnp.dot(a, b, preferred_element_type=jnp.float32)


def _vit_kernel(patches_ref, mem_k_ref, mem_v_ref, patch_w_ref, patch_b_ref,
                pos_ref, ln1_g_ref, ln1_b_ref, wq_ref, wk_ref, wv_ref, wo_ref,
                ln2_g_ref, ln2_b_ref, w1_ref, b1_ref, w2_ref, b2_ref,
                lnf_g_ref, lnf_b_ref, head_w_ref, head_b_ref,
                logits_ref, feats_ref):
  for bi in range(BB):
    x = (_dot(patches_ref[bi], patch_w_ref[...]) + patch_b_ref[...]
         + pos_ref[...])
    scale = D ** -0.5
    for i in range(DEPTH):
        h = _ln(x, ln1_g_ref[i], ln1_b_ref[i])
        q = _dot(h, wq_ref[i])
        k = _dot(h, wk_ref[i])
        v = _dot(h, wv_ref[i])
        local = jax.lax.dot_general(
            q, k, (((1,), (1,)), ((), ())),
            preferred_element_type=jnp.float32) * scale
        if i == MEM_LAYER:
            mem_k = mem_k_ref[bi]
            mem_v = mem_v_ref[bi]
            sims = jax.lax.dot_general(
                q, mem_k, (((1,), (1,)), ((), ())),
                preferred_element_type=jnp.float32)

            L = 8
            neg = jnp.full((NT, 128), -jnp.inf, jnp.float32)
            tops = [sims[:, :128]] + [neg] * (L - 1)
            for c in range(1, M // 128):
                cand = sims[:, c * 128:(c + 1) * 128]
                for l in range(L):
                    hi = jnp.maximum(tops[l], cand)
                    cand = jnp.minimum(tops[l], cand)
                    tops[l] = hi
            heads0 = tops[0]

            lomin = tops[L - 1]
            for l in range(L - 1):
                lomin = jnp.minimum(lomin, tops[l])
            lo0 = lomin.min(-1, keepdims=True)
            hi0 = heads0.max(-1, keepdims=True)

            def _count(t):
                s = jnp.zeros((NT, 128), jnp.float32)
                for l in range(L):
                    s = s + (tops[l] >= t).astype(jnp.float32)
                return s.sum(-1, keepdims=True)

            def bis_body(_, carry):
                lo, hi = carry
                mid = 0.5 * (lo + hi)
                take = _count(mid) >= float(K)
                return (jnp.where(take, mid, lo),
                        jnp.where(take, hi, mid))

            lo_f, _ = jax.lax.fori_loop(0, 26, bis_body, (lo0, hi0))
            thr_fast = lo_f
            ok = jnp.all(_count(thr_fast) == float(K)) & jnp.all(
                tops[L - 1] < thr_fast)

            def slow_path(_):
                def body(_, t):
                    masked = jnp.where(sims < t, sims, -jnp.inf)
                    return masked.max(-1, keepdims=True)
                t0 = jnp.full((NT, 1), jnp.inf, jnp.float32)
                return jax.lax.fori_loop(0, K, body, t0)

            thr = jax.lax.cond(ok, lambda _: thr_fast, slow_path, None)

            mmax = heads0.max(-1, keepdims=True) * scale
            lmax = local.max(-1, keepdims=True)
            rowmax = jnp.maximum(lmax, mmax)
            lexp = jnp.exp(local - rowmax)
            mexp = jnp.where(sims >= thr,
                             jnp.exp(sims * scale - rowmax),
                             jnp.float32(0.0))
            denom = lexp.sum(-1, keepdims=True) + mexp.sum(-1, keepdims=True)
            out = (_dot(lexp, v) + _dot(mexp, mem_v)) / denom
        else:
            lmax = local.max(-1, keepdims=True)
            lexp = jnp.exp(local - lmax)
            out = _dot(lexp, v) / lexp.sum(-1, keepdims=True)
        x = x + _dot(out, wo_ref[i])
        h = _ln(x, ln2_g_ref[i], ln2_b_ref[i])
        ff = jax.nn.gelu(_dot(h, w1_ref[i]) + b1_ref[i])
        x = x + _dot(ff, w2_ref[i]) + b2_ref[i]
    feats = _ln(x, lnf_g_ref[...], lnf_b_ref[...]).mean(0, keepdims=True)
    logits = _dot(feats, head_w_ref[...]) + head_b_ref[...]
    logits_ref[bi] = logits
    feats_ref[bi] = feats


def kernel(image, patch_w, patch_b, pos, ln1_g, ln1_b, wq, wk, wv, wo,
           ln2_g, ln2_b, w1, b1, w2, b2, lnf_g, lnf_b, head_w, head_b,
           mem_k, mem_v):
    patches = image.reshape(B, 3, G, P, G, P).transpose(0, 2, 4, 1, 3, 5)
    patches = patches.reshape(B, NT, 3 * P * P)

    patch_b2 = patch_b.reshape(1, D)
    lnf_g2 = lnf_g.reshape(1, D)
    lnf_b2 = lnf_b.reshape(1, D)
    head_b2 = head_b.reshape(1, NC)

    def batch_spec(shape):
        return pl.BlockSpec((BB,) + shape, lambda b: (b,) + (0,) * len(shape))

    def full_spec(shape):
        return pl.BlockSpec(shape, lambda b: (0,) * len(shape))

    out = pl.pallas_call(
        _vit_kernel,
        grid=(B // BB,),
        in_specs=[
            batch_spec((NT, 3 * P * P)),
            batch_spec((M, D)),
            batch_spec((M, D)),
            full_spec((3 * P * P, D)),
            full_spec((1, D)),
            full_spec((NT, D)),
            full_spec((DEPTH, D)),
            full_spec((DEPTH, D)),
            full_spec((DEPTH, D, D)),
            full_spec((DEPTH, D, D)),
            full_spec((DEPTH, D, D)),
            full_spec((DEPTH, D, D)),
            full_spec((DEPTH, D)),
            full_spec((DEPTH, D)),
            full_spec((DEPTH, D, FF)),
            full_spec((DEPTH, FF)),
            full_spec((DEPTH, FF, D)),
            full_spec((DEPTH, D)),
            full_spec((1, D)),
            full_spec((1, D)),
            full_spec((D, NC)),
            full_spec((1, NC)),
        ],
        out_specs=[
            pl.BlockSpec((BB, 1, NC), lambda b: (b, 0, 0)),
            pl.BlockSpec((BB, 1, D), lambda b: (b, 0, 0)),
        ],
        out_shape=[
            jax.ShapeDtypeStruct((B, 1, NC), jnp.float32),
            jax.ShapeDtypeStruct((B, 1, D), jnp.float32),
        ],
        compiler_params=pltpu.CompilerParams(
            dimension_semantics=("arbitrary",)),
    )(patches, mem_k, mem_v, patch_w, patch_b2, pos, ln1_g, ln1_b,
      wq, wk, wv, wo, ln2_g, ln2_b, w1, b1, w2, b2, lnf_g2, lnf_b2,
      head_w, head_b2)
    logits, feats = out
    return logits.reshape(B, NC), feats.reshape(B, D)


# speedup vs baseline: 1.0118x; 1.0118x over previous
import functools

import jax
import jax.numpy as jnp
from jax.experimental import pallas as pl
from jax.experimental.pallas import tpu as pltpu

B = 32
D = 128
FF = 512
DEPTH = 7
MEM_LAYER = 6
M = 8192
K = 32
P = 16
G = 224 // P
NT = G * G
NC = 1000


def _ln(x, g, b):
    m = x.mean(-1, keepdims=True)
    v = ((x - m) ** 2).mean(-1, keepdims=True)
    return (x - m) * jax.lax.rsqrt(v + 1e-5) * g + b


def _dot(a, b):
    return jnp.dot(a, b, preferred_element_type=jnp.float32)


def _vit_kernel(patches_ref, mem_k_ref, mem_v_ref, patch_w_ref, patch_b_ref,
                pos_ref, ln1_g_ref, ln1_b_ref, wq_ref, wk_ref, wv_ref, wo_ref,
                ln2_g_ref, ln2_b_ref, w1_ref, b1_ref, w2_ref, b2_ref,
                lnf_g_ref, lnf_b_ref, head_w_ref, head_b_ref,
                logits_ref, feats_ref):
    x = _dot(patches_ref[0], patch_w_ref[...]) + patch_b_ref[...] + pos_ref[...]
    scale = D ** -0.5
    for i in range(DEPTH):
        h = _ln(x, ln1_g_ref[i], ln1_b_ref[i])
        q = _dot(h, wq_ref[i])
        k = _dot(h, wk_ref[i])
        v = _dot(h, wv_ref[i])
        local = jax.lax.dot_general(
            q, k, (((1,), (1,)), ((), ())),
            preferred_element_type=jnp.float32) * scale
        if i == MEM_LAYER:
            mem_k = mem_k_ref[0]
            mem_v = mem_v_ref[0]
            sims = jax.lax.dot_general(
                q, mem_k, (((1,), (1,)), ((), ())),
                preferred_element_type=jnp.float32)

            L = 8
            neg = jnp.full((NT, 128), -jnp.inf, jnp.float32)
            tops = [sims[:, :128]] + [neg] * (L - 1)
            for c in range(1, M // 128):
                cand = sims[:, c * 128:(c + 1) * 128]
                for l in range(L):
                    hi = jnp.maximum(tops[l], cand)
                    cand = jnp.minimum(tops[l], cand)
                    tops[l] = hi
            heads0 = tops[0]

            def ext_body(_, carry):
                h, vl, _t = carry
                t_j = h.max(-1, keepdims=True)
                adv = h >= t_j
                vln = vl + adv.astype(jnp.float32)
                nxt = jnp.full_like(h, -jnp.inf)
                for l in range(L - 1, 0, -1):
                    nxt = jnp.where(vln == l, tops[l], nxt)
                hn = jnp.where(adv, nxt, h)
                return hn, vln, t_j

            vl0 = jnp.zeros((NT, 128), jnp.float32)
            t00 = jnp.zeros((NT, 1), jnp.float32)
            _, vl_f, thr_fast = jax.lax.fori_loop(
                0, K, ext_body, (tops[0], vl0, t00))
            ok = vl_f.max() < float(L)

            def slow_path(_):
                def body(_, t):
                    masked = jnp.where(sims < t, sims, -jnp.inf)
                    return masked.max(-1, keepdims=True)
                t0 = jnp.full((NT, 1), jnp.inf, jnp.float32)
                return jax.lax.fori_loop(0, K, body, t0)

            thr = jax.lax.cond(ok, lambda _: thr_fast, slow_path, None)

            mmax = heads0.max(-1, keepdims=True) * scale
            lmax = local.max(-1, keepdims=True)
            rowmax = jnp.maximum(lmax, mmax)
            lexp = jnp.exp(local - rowmax)
            mexp = jnp.where(sims >= thr,
                             jnp.exp(sims * scale - rowmax),
                             jnp.float32(0.0))
            denom = lexp.sum(-1, keepdims=True) + mexp.sum(-1, keepdims=True)
            out = (_dot(lexp, v) + _dot(mexp, mem_v)) / denom
        else:
            lmax = local.max(-1, keepdims=True)
            lexp = jnp.exp(local - lmax)
            out = _dot(lexp, v) / lexp.sum(-1, keepdims=True)
        x = x + _dot(out, wo_ref[i])
        h = _ln(x, ln2_g_ref[i], ln2_b_ref[i])
        ff = jax.nn.gelu(_dot(h, w1_ref[i]) + b1_ref[i])
        x = x + _dot(ff, w2_ref[i]) + b2_ref[i]
    feats = _ln(x, lnf_g_ref[...], lnf_b_ref[...]).mean(0, keepdims=True)
    logits = _dot(feats, head_w_ref[...]) + head_b_ref[...]
    logits_ref[0] = logits
    feats_ref[0] = feats


def kernel(image, patch_w, patch_b, pos, ln1_g, ln1_b, wq, wk, wv, wo,
           ln2_g, ln2_b, w1, b1, w2, b2, lnf_g, lnf_b, head_w, head_b,
           mem_k, mem_v):
    patches = image.reshape(B, 3, G, P, G, P).transpose(0, 2, 4, 1, 3, 5)
    patches = patches.reshape(B, NT, 3 * P * P)

    patch_b2 = patch_b.reshape(1, D)
    lnf_g2 = lnf_g.reshape(1, D)
    lnf_b2 = lnf_b.reshape(1, D)
    head_b2 = head_b.reshape(1, NC)

    def batch_spec(shape):
        return pl.BlockSpec((1,) + shape, lambda b: (b,) + (0,) * len(shape))

    def full_spec(shape):
        return pl.BlockSpec(shape, lambda b: (0,) * len(shape))

    out = pl.pallas_call(
        _vit_kernel,
        grid=(B,),
        in_specs=[
            batch_spec((NT, 3 * P * P)),
            batch_spec((M, D)),
            batch_spec((M, D)),
            full_spec((3 * P * P, D)),
            full_spec((1, D)),
            full_spec((NT, D)),
            full_spec((DEPTH, D)),
            full_spec((DEPTH, D)),
            full_spec((DEPTH, D, D)),
            full_spec((DEPTH, D, D)),
            full_spec((DEPTH, D, D)),
            full_spec((DEPTH, D, D)),
            full_spec((DEPTH, D)),
            full_spec((DEPTH, D)),
            full_spec((DEPTH, D, FF)),
            full_spec((DEPTH, FF)),
            full_spec((DEPTH, FF, D)),
            full_spec((DEPTH, D)),
            full_spec((1, D)),
            full_spec((1, D)),
            full_spec((D, NC)),
            full_spec((1, NC)),
        ],
        out_specs=[
            pl.BlockSpec((1, 1, NC), lambda b: (b, 0, 0)),
            pl.BlockSpec((1, 1, D), lambda b: (b, 0, 0)),
        ],
        out_shape=[
            jax.ShapeDtypeStruct((B, 1, NC), jnp.float32),
            jax.ShapeDtypeStruct((B, 1, D), jnp.float32),
        ],
        compiler_params=pltpu.CompilerParams(
            dimension_semantics=("arbitrary",)),
    )(patches, mem_k, mem_v, patch_w, patch_b2, pos, ln1_g, ln1_b,
      wq, wk, wv, wo, ln2_g, ln2_b, w1, b1, w2, b2, lnf_g2, lnf_b2,
      head_w, head_b2)
    logits, feats = out
    return logits.reshape(B, NC), feats.reshape(B, D)
